# baseline (speedup 1.0000x reference)
import functools

import jax
import jax.numpy as jnp
from jax import lax
from jax.experimental import pallas as pl
from jax.experimental.pallas import tpu as pltpu
from jax.experimental.pallas import tpu_sc as plsc

T = 512
CH = 3
H = 224
W = 224
NKEY = 171
NW = 32
NS = CH * H
SPT = NS // NW
NCH = 7
CW = W // NCH


def kernel(video):
    v3 = jnp.transpose(video, (1, 2, 3, 0)).reshape(NS, W, T)
    mesh = plsc.VectorSubcoreMesh(core_axis_name="c", subcore_axis_name="s")

    @functools.partial(
        pl.kernel,
        mesh=mesh,
        out_type=jax.ShapeDtypeStruct((NKEY, CH, H, W), jnp.float32),
        scratch_types=(
            [pltpu.VMEM((CW, T), jnp.float32)] * 2
            + [pltpu.VMEM((NKEY, W), jnp.float32)] * 2
            + [pltpu.SemaphoreType.DMA] * 4
        ),
        compiler_params=pltpu.CompilerParams(
            use_tc_tiling_on_sc=True, needs_layout_passes=False),
    )
    def k(v_hbm, o_hbm, ib0, ib1, ob0, ob1, *sems):
        inbufs = (ib0, ib1)
        outbufs = (ob0, ob1)
        gsems = sems[0:2]
        ssems = sems[2:4]
        wid = lax.axis_index("s") * 2 + lax.axis_index("c")
        w16 = lax.iota(jnp.int32, 16)

        def stripe_id(t):
            return t * NW + wid

        def in_copy(t, k_, i):
            return pltpu.make_async_copy(
                v_hbm.at[stripe_id(t), pl.ds(k_ * CW, CW)],
                inbufs[i % 2],
                gsems[i % 2],
            )

        def out_copy(t):
            s = stripe_id(t)
            c = s // H
            h = s % H
            return pltpu.make_async_copy(
                outbufs[t % 2], o_hbm.at[:, c, h], ssems[t % 2])

        def gather_chunk(t, k_, i):
            inb = inbufs[i % 2]
            outb = outbufs[t % 2]

            def fbody(f, _):
                src = jnp.full((16,), jnp.maximum(3 * f - 1, 0), jnp.int32)
                v0 = plsc.load_gather(inb, [w16, src])
                v1 = plsc.load_gather(inb, [w16 + 16, src])
                outb[f, pl.ds(k_ * CW, 16)] = v0
                outb[f, pl.ds(k_ * CW + 16, 16)] = v1
                return 0

            lax.fori_loop(0, NKEY, fbody, 0)

        in_copy(0, 0, 0).start()
        for t in range(SPT):
            for k_ in range(NCH):
                i = t * NCH + k_
                if i + 1 < SPT * NCH:
                    nt, nk = divmod(i + 1, NCH)
                    in_copy(nt, nk, i + 1).start()
                if k_ == 0 and t >= 2:
                    out_copy(t - 2).wait()
                in_copy(t, k_, i).wait()
                gather_chunk(t, k_, i)
            out_copy(t).start()
        for t in (SPT - 2, SPT - 1):
            out_copy(t).wait()

    return k(v3)
